# baseline (speedup 1.0000x reference)
import functools

import jax
import jax.numpy as jnp
from jax import lax
from jax.experimental import pallas as pl
from jax.experimental.pallas import tpu as pltpu
from jax.experimental.pallas import tpu_sc as plsc

NUM_CORES = 2
NUM_SUBCORES = 16
NUM_WORKERS = NUM_CORES * NUM_SUBCORES
LANES = 16

ROWS = 1024
COLS = 100000
K = 200

TOTAL = ROWS * K
CHUNK = TOTAL // NUM_WORKERS
ROWS_PER_W = ROWS // NUM_WORKERS
DMA_IDX = 128
N_DMA = CHUNK // DMA_IDX
GROUP = 10
N_GROUP = N_DMA // GROUP
FULL_SLICES = K // LANES
TAIL_START = K - LANES


def _sc_gather(flat_tensor, flat_index):
  mesh = plsc.VectorSubcoreMesh(
      core_axis_name="c", subcore_axis_name="s",
      num_cores=NUM_CORES, num_subcores=NUM_SUBCORES)

  @functools.partial(
      pl.kernel,
      out_type=jax.ShapeDtypeStruct((TOTAL,), jnp.float32),
      mesh=mesh,
      scratch_types=[
          pltpu.VMEM((CHUNK,), jnp.int32),
          pltpu.VMEM((CHUNK,), jnp.int32),
          pltpu.VMEM((CHUNK,), jnp.float32),
          pltpu.SemaphoreType.DMA,
      ],
  )
  def k(tensor_hbm, idx_hbm, out_hbm, idx_v, gidx_v, data_v, sem):
    wid = lax.axis_index("s") * NUM_CORES + lax.axis_index("c")
    base = wid * CHUNK
    pltpu.sync_copy(idx_hbm.at[pl.ds(base, CHUNK)], idx_v)

    row0 = wid * ROWS_PER_W

    def row_body(r, carry):
      off_vec = jnp.zeros((LANES,), jnp.int32) + (row0 + r) * COLS
      rbase = r * K
      for t in range(FULL_SLICES):
        s = rbase + t * LANES
        gidx_v[pl.ds(s, LANES)] = idx_v[pl.ds(s, LANES)] + off_vec
      s = rbase + TAIL_START
      gidx_v[pl.ds(s, LANES)] = idx_v[pl.ds(s, LANES)] + off_vec
      return carry

    lax.fori_loop(0, ROWS_PER_W, row_body, 0)

    def grp_body(g, carry):
      c0 = g * GROUP
      copies = []
      for b in range(GROUP):
        start = (c0 + b) * DMA_IDX
        cp = pltpu.make_async_copy(
            tensor_hbm.at[gidx_v.at[pl.ds(start, DMA_IDX)]],
            data_v.at[pl.ds(start, DMA_IDX)],
            sem)
        cp.start()
        copies.append(cp)
      for cp in copies:
        cp.wait()
      return carry

    lax.fori_loop(0, N_GROUP, grp_body, 0)

    pltpu.sync_copy(data_v, out_hbm.at[pl.ds(base, CHUNK)])

  return k(flat_tensor, flat_index)


def kernel(tensor, index):
  out = _sc_gather(tensor.reshape(-1), index.reshape(-1))
  return out.reshape(ROWS, K)


# speedup vs baseline: 26.1005x; 26.1005x over previous
import functools

import jax
import jax.numpy as jnp
from jax import lax
from jax.experimental import pallas as pl
from jax.experimental.pallas import tpu as pltpu
from jax.experimental.pallas import tpu_sc as plsc

NUM_CORES = 2
NUM_SUBCORES = 16
NUM_WORKERS = NUM_CORES * NUM_SUBCORES
LANES = 16

ROWS = 1024
COLS = 100000
K = 200

TOTAL = ROWS * K
CHUNK = TOTAL // NUM_WORKERS
VREGS = CHUNK // LANES
DMA_IDX = 128
N_DMA = CHUNK // DMA_IDX
GROUP = 10
N_GROUP = N_DMA // GROUP


def _to_phys(x):
  r, c = x.shape
  return x.reshape(r // 128, 128, c // 8, 8).transpose(2, 0, 3, 1).reshape(-1)


def _from_phys(flat, r, c):
  return (flat.reshape(c // 8, r // 128, 8, 128)
          .transpose(1, 3, 0, 2).reshape(r, c))


def _sc_gather(flat_tensor, flat_index):
  mesh = plsc.VectorSubcoreMesh(
      core_axis_name="c", subcore_axis_name="s",
      num_cores=NUM_CORES, num_subcores=NUM_SUBCORES)

  @functools.partial(
      pl.kernel,
      out_type=jax.ShapeDtypeStruct((TOTAL,), jnp.float32),
      mesh=mesh,
      scratch_types=[
          pltpu.VMEM((CHUNK,), jnp.int32),
          pltpu.VMEM((CHUNK,), jnp.int32),
          pltpu.VMEM((CHUNK,), jnp.float32),
          pltpu.SemaphoreType.DMA,
      ],
  )
  def k(tensor_hbm, idx_hbm, out_hbm, idx_v, gidx_v, data_v, sem):
    wid = lax.axis_index("s") * NUM_CORES + lax.axis_index("c")
    base = wid * CHUNK
    pltpu.sync_copy(idx_hbm.at[pl.ds(base, CHUNK)], idx_v)

    lane = lax.iota(jnp.int32, LANES)

    def vreg_body(t, carry):
      s = t * LANES
      n_base = base + s
      row_part = ((lax.shift_right_logical(n_base, 10) & 7) << 10) \
          + (n_base & 127)
      ix = idx_v[pl.ds(s, LANES)]
      m = (lax.shift_right_logical(ix, 3) << 13) \
          + ((ix & 7) << 7) + (row_part + lane)
      gidx_v[pl.ds(s, LANES)] = m
      return carry

    lax.fori_loop(0, VREGS, vreg_body, 0)

    def grp_body(g, carry):
      c0 = g * GROUP
      copies = []
      for b in range(GROUP):
        start = (c0 + b) * DMA_IDX
        cp = pltpu.make_async_copy(
            tensor_hbm.at[gidx_v.at[pl.ds(start, DMA_IDX)]],
            data_v.at[pl.ds(start, DMA_IDX)],
            sem)
        cp.start()
        copies.append(cp)
      for cp in copies:
        cp.wait()
      return carry

    lax.fori_loop(0, N_GROUP, grp_body, 0)

    pltpu.sync_copy(data_v, out_hbm.at[pl.ds(base, CHUNK)])

  return k(flat_tensor, flat_index)


def kernel(tensor, index):
  out = _sc_gather(_to_phys(tensor), _to_phys(index))
  return _from_phys(out, ROWS, K)


# speedup vs baseline: 26.8756x; 1.0297x over previous
import functools

import jax
import jax.numpy as jnp
from jax import lax
from jax.experimental import pallas as pl
from jax.experimental.pallas import tpu as pltpu
from jax.experimental.pallas import tpu_sc as plsc

NUM_CORES = 2
NUM_SUBCORES = 16
NUM_WORKERS = NUM_CORES * NUM_SUBCORES
LANES = 16

ROWS = 1024
COLS = 100000
K = 200

TOTAL = ROWS * K
CHUNK = TOTAL // NUM_WORKERS
DMA_IDX = 128
N_DMA = CHUNK // DMA_IDX
VREGS_PER_DMA = DMA_IDX // LANES
DEPTH = 16


def _to_phys(x):
  r, c = x.shape
  return x.reshape(r // 128, 128, c // 8, 8).transpose(2, 0, 3, 1).reshape(-1)


def _from_phys(flat, r, c):
  return (flat.reshape(c // 8, r // 128, 8, 128)
          .transpose(1, 3, 0, 2).reshape(r, c))


def _sc_gather(flat_tensor, flat_index):
  mesh = plsc.VectorSubcoreMesh(
      core_axis_name="c", subcore_axis_name="s",
      num_cores=NUM_CORES, num_subcores=NUM_SUBCORES)

  @functools.partial(
      pl.kernel,
      out_type=jax.ShapeDtypeStruct((NUM_WORKERS, N_DMA, DMA_IDX),
                                    jnp.float32),
      mesh=mesh,
      scratch_types=[
          pltpu.VMEM((CHUNK,), jnp.int32),
          pltpu.VMEM((N_DMA, DMA_IDX), jnp.int32),
          pltpu.VMEM((N_DMA, DMA_IDX), jnp.float32),
          pltpu.SemaphoreType.DMA,
      ],
  )
  def k(tensor_hbm, idx_hbm, out_hbm, idx_v, gidx_v, data_v, sem):
    wid = lax.axis_index("s") * NUM_CORES + lax.axis_index("c")
    base = wid * CHUNK
    pltpu.sync_copy(idx_hbm.at[pl.ds(base, CHUNK)], idx_v)

    lane = lax.iota(jnp.int32, LANES)

    def compute_and_fire(c):
      s0 = c * DMA_IDX
      for u in range(VREGS_PER_DMA):
        s = s0 + u * LANES
        n_base = base + s
        row_part = ((lax.shift_right_logical(n_base, 10) & 7) << 10) \
            + (n_base & 127)
        ix = idx_v[pl.ds(s, LANES)]
        m = (lax.shift_right_logical(ix, 3) << 13) \
            + ((ix & 7) << 7) + (row_part + lane)
        gidx_v[c, pl.ds(u * LANES, LANES)] = m
      pltpu.make_async_copy(
          tensor_hbm.at[gidx_v.at[c]], data_v.at[c], sem).start()

    def wait_one():
      pltpu.make_async_copy(
          tensor_hbm.at[gidx_v.at[0]], data_v.at[0], sem).wait()

    def fill_body(c, carry):
      compute_and_fire(c)
      return carry

    def steady_body(c, carry):
      wait_one()
      compute_and_fire(c)
      return carry

    def drain_body(c, carry):
      wait_one()
      return carry

    lax.fori_loop(0, DEPTH, fill_body, 0)
    lax.fori_loop(DEPTH, N_DMA, steady_body, 0)
    lax.fori_loop(0, DEPTH, drain_body, 0)

    pltpu.sync_copy(data_v, out_hbm.at[wid])

  return k(flat_tensor, flat_index)


def kernel(tensor, index):
  out = _sc_gather(_to_phys(tensor), _to_phys(index))
  return _from_phys(out.reshape(TOTAL), ROWS, K)


# speedup vs baseline: 27.0684x; 1.0072x over previous
import functools

import jax
import jax.numpy as jnp
from jax import lax
from jax.experimental import pallas as pl
from jax.experimental.pallas import tpu as pltpu
from jax.experimental.pallas import tpu_sc as plsc

NUM_CORES = 2
NUM_SUBCORES = 16
NUM_WORKERS = NUM_CORES * NUM_SUBCORES
LANES = 16

ROWS = 1024
COLS = 100000
K = 200

TOTAL = ROWS * K
CHUNK = TOTAL // NUM_WORKERS
DMA_IDX = 128
N_DMA = CHUNK // DMA_IDX
VREGS_PER_DMA = DMA_IDX // LANES
DEPTH = 16


def _to_phys(x):
  r, c = x.shape
  return x.reshape(r // 128, 128, c // 8, 8).transpose(2, 0, 3, 1).reshape(-1)


def _from_phys(flat, r, c):
  return (flat.reshape(c // 8, r // 128, 8, 128)
          .transpose(1, 3, 0, 2).reshape(r, c))


def _sc_gather(flat_tensor, flat_index):
  mesh = plsc.VectorSubcoreMesh(
      core_axis_name="c", subcore_axis_name="s",
      num_cores=NUM_CORES, num_subcores=NUM_SUBCORES)

  @functools.partial(
      pl.kernel,
      out_type=jax.ShapeDtypeStruct((NUM_WORKERS, N_DMA, DMA_IDX),
                                    jnp.float32),
      mesh=mesh,
      scratch_types=[
          pltpu.VMEM((CHUNK,), jnp.int32),
          pltpu.VMEM((N_DMA, DMA_IDX), jnp.int32),
          pltpu.VMEM((N_DMA, DMA_IDX), jnp.float32),
          pltpu.SemaphoreType.DMA,
      ],
  )
  def k(tensor_hbm, idx_hbm, out_hbm, idx_v, gidx_v, data_v, sem):
    wid = lax.axis_index("s") * NUM_CORES + lax.axis_index("c")
    base = wid * CHUNK
    pltpu.sync_copy(idx_hbm.at[pl.ds(base, CHUNK)], idx_v)

    lane = lax.iota(jnp.int32, LANES)

    def compute_and_fire(c):
      s0 = c * DMA_IDX
      ixs = [idx_v[pl.ds(s0 + u * LANES, LANES)]
             for u in range(VREGS_PER_DMA)]
      ms = []
      for u in range(VREGS_PER_DMA):
        n_base = base + s0 + u * LANES
        row_part = ((lax.shift_right_logical(n_base, 10) & 7) << 10) \
            + (n_base & 127)
        ix = ixs[u]
        ms.append((lax.shift_right_logical(ix, 3) << 13)
                  + ((ix & 7) << 7) + (row_part + lane))
      for u in range(VREGS_PER_DMA):
        gidx_v[c, pl.ds(u * LANES, LANES)] = ms[u]
      pltpu.make_async_copy(
          tensor_hbm.at[gidx_v.at[c]], data_v.at[c], sem).start()

    def wait_one():
      pltpu.make_async_copy(
          tensor_hbm.at[gidx_v.at[0]], data_v.at[0], sem).wait()

    def fill_body(c, carry):
      compute_and_fire(c)
      return carry

    def steady_body(c, carry):
      wait_one()
      compute_and_fire(c)
      return carry

    def drain_body(c, carry):
      wait_one()
      return carry

    lax.fori_loop(0, DEPTH, fill_body, 0)
    lax.fori_loop(DEPTH, N_DMA, steady_body, 0)
    lax.fori_loop(0, DEPTH, drain_body, 0)

    pltpu.sync_copy(data_v, out_hbm.at[wid])

  return k(flat_tensor, flat_index)


def kernel(tensor, index):
  out = _sc_gather(_to_phys(tensor), _to_phys(index))
  return _from_phys(out.reshape(TOTAL), ROWS, K)


# speedup vs baseline: 27.2212x; 1.0056x over previous
import functools

import jax
import jax.numpy as jnp
from jax import lax
from jax.experimental import pallas as pl
from jax.experimental.pallas import tpu as pltpu
from jax.experimental.pallas import tpu_sc as plsc

NUM_CORES = 2
NUM_SUBCORES = 16
NUM_WORKERS = NUM_CORES * NUM_SUBCORES
LANES = 16

ROWS = 1024
COLS = 100000
K = 200

TOTAL = ROWS * K
CHUNK = TOTAL // NUM_WORKERS
DMA_IDX = 128
N_DMA = CHUNK // DMA_IDX
VREGS_PER_DMA = DMA_IDX // LANES
DEPTH = 16


def _to_phys(x):
  r, c = x.shape
  return x.reshape(r // 128, 128, c // 8, 8).transpose(2, 0, 3, 1).reshape(-1)


def _from_phys(flat, r, c):
  return (flat.reshape(c // 8, r // 128, 8, 128)
          .transpose(1, 3, 0, 2).reshape(r, c))


def _sc_gather(flat_tensor, flat_index):
  mesh = plsc.VectorSubcoreMesh(
      core_axis_name="c", subcore_axis_name="s",
      num_cores=NUM_CORES, num_subcores=NUM_SUBCORES)

  @functools.partial(
      pl.kernel,
      out_type=jax.ShapeDtypeStruct((NUM_WORKERS, N_DMA, DMA_IDX),
                                    jnp.float32),
      mesh=mesh,
      scratch_types=[
          pltpu.VMEM((CHUNK,), jnp.int32),
          pltpu.VMEM((N_DMA, DMA_IDX), jnp.int32),
          pltpu.VMEM((N_DMA, DMA_IDX), jnp.float32),
          pltpu.SemaphoreType.DMA,
          pltpu.SemaphoreType.DMA,
      ],
  )
  def k(tensor_hbm, idx_hbm, out_hbm, idx_v, gidx_v, data_v, sem, sem2):
    wid = lax.axis_index("s") * NUM_CORES + lax.axis_index("c")
    base = wid * CHUNK
    split = DEPTH * DMA_IDX
    stage_a = pltpu.make_async_copy(
        idx_hbm.at[pl.ds(base, split)], idx_v.at[pl.ds(0, split)], sem2)
    stage_b = pltpu.make_async_copy(
        idx_hbm.at[pl.ds(base + split, CHUNK - split)],
        idx_v.at[pl.ds(split, CHUNK - split)], sem2)
    stage_a.start()
    stage_b.start()
    stage_a.wait()

    lane = lax.iota(jnp.int32, LANES)

    def compute_and_fire(c):
      s0 = c * DMA_IDX
      ixs = [idx_v[pl.ds(s0 + u * LANES, LANES)]
             for u in range(VREGS_PER_DMA)]
      ms = []
      for u in range(VREGS_PER_DMA):
        n_base = base + s0 + u * LANES
        row_part = ((lax.shift_right_logical(n_base, 10) & 7) << 10) \
            + (n_base & 127)
        ix = ixs[u]
        ms.append((lax.shift_right_logical(ix, 3) << 13)
                  + ((ix & 7) << 7) + (row_part + lane))
      for u in range(VREGS_PER_DMA):
        gidx_v[c, pl.ds(u * LANES, LANES)] = ms[u]
      pltpu.make_async_copy(
          tensor_hbm.at[gidx_v.at[c]], data_v.at[c], sem).start()

    def wait_one():
      pltpu.make_async_copy(
          tensor_hbm.at[gidx_v.at[0]], data_v.at[0], sem).wait()

    def fill_body(c, carry):
      compute_and_fire(c)
      return carry

    def steady_body(c, carry):
      wait_one()
      compute_and_fire(c)
      return carry

    def drain_body(c, carry):
      wait_one()
      return carry

    lax.fori_loop(0, DEPTH, fill_body, 0)
    stage_b.wait()
    lax.fori_loop(DEPTH, N_DMA, steady_body, 0)
    lax.fori_loop(0, DEPTH, drain_body, 0)

    pltpu.sync_copy(data_v, out_hbm.at[wid])

  return k(flat_tensor, flat_index)


def kernel(tensor, index):
  out = _sc_gather(_to_phys(tensor), _to_phys(index))
  return _from_phys(out.reshape(TOTAL), ROWS, K)


# speedup vs baseline: 27.7953x; 1.0211x over previous
import functools

import jax
import jax.numpy as jnp
from jax import lax
from jax.experimental import pallas as pl
from jax.experimental.pallas import tpu as pltpu
from jax.experimental.pallas import tpu_sc as plsc

NUM_CORES = 2
NUM_SUBCORES = 16
NUM_WORKERS = NUM_CORES * NUM_SUBCORES
LANES = 16

ROWS = 1024
COLS = 100000
K = 200

TOTAL = ROWS * K
CHUNK = TOTAL // NUM_WORKERS
DMA_IDX = 128
N_DMA = CHUNK // DMA_IDX
VREGS_PER_DMA = DMA_IDX // LANES
DEPTH = 32


def _to_phys(x):
  r, c = x.shape
  return x.reshape(r // 128, 128, c // 8, 8).transpose(2, 0, 3, 1).reshape(-1)


def _from_phys(flat, r, c):
  return (flat.reshape(c // 8, r // 128, 8, 128)
          .transpose(1, 3, 0, 2).reshape(r, c))


def _sc_gather(flat_tensor, flat_index):
  mesh = plsc.VectorSubcoreMesh(
      core_axis_name="c", subcore_axis_name="s",
      num_cores=NUM_CORES, num_subcores=NUM_SUBCORES)

  @functools.partial(
      pl.kernel,
      out_type=jax.ShapeDtypeStruct((NUM_WORKERS, N_DMA, DMA_IDX),
                                    jnp.float32),
      mesh=mesh,
      scratch_types=[
          pltpu.VMEM((CHUNK,), jnp.int32),
          pltpu.VMEM((N_DMA, DMA_IDX), jnp.int32),
          pltpu.VMEM((N_DMA, DMA_IDX), jnp.float32),
          pltpu.SemaphoreType.DMA,
          pltpu.SemaphoreType.DMA,
      ],
  )
  def k(tensor_hbm, idx_hbm, out_hbm, idx_v, gidx_v, data_v, sem, sem2):
    wid = lax.axis_index("s") * NUM_CORES + lax.axis_index("c")
    base = wid * CHUNK
    split = DEPTH * DMA_IDX
    stage_a = pltpu.make_async_copy(
        idx_hbm.at[pl.ds(base, split)], idx_v.at[pl.ds(0, split)], sem2)
    stage_b = pltpu.make_async_copy(
        idx_hbm.at[pl.ds(base + split, CHUNK - split)],
        idx_v.at[pl.ds(split, CHUNK - split)], sem2)
    stage_a.start()
    stage_b.start()
    stage_a.wait()

    lane = lax.iota(jnp.int32, LANES)

    def compute_and_fire(c):
      s0 = c * DMA_IDX
      ixs = [idx_v[pl.ds(s0 + u * LANES, LANES)]
             for u in range(VREGS_PER_DMA)]
      ms = []
      for u in range(VREGS_PER_DMA):
        n_base = base + s0 + u * LANES
        row_part = ((lax.shift_right_logical(n_base, 10) & 7) << 10) \
            + (n_base & 127)
        ix = ixs[u]
        ms.append((lax.shift_right_logical(ix, 3) << 13)
                  + ((ix & 7) << 7) + (row_part + lane))
      for u in range(VREGS_PER_DMA):
        gidx_v[c, pl.ds(u * LANES, LANES)] = ms[u]
      pltpu.make_async_copy(
          tensor_hbm.at[gidx_v.at[c]], data_v.at[c], sem).start()

    def wait_one():
      pltpu.make_async_copy(
          tensor_hbm.at[gidx_v.at[0]], data_v.at[0], sem).wait()

    def fill_body(c, carry):
      compute_and_fire(c)
      return carry

    def steady_body(c, carry):
      wait_one()
      compute_and_fire(c)
      return carry

    def drain_body(c, carry):
      wait_one()
      return carry

    lax.fori_loop(0, DEPTH, fill_body, 0)
    stage_b.wait()
    lax.fori_loop(DEPTH, N_DMA, steady_body, 0)
    lax.fori_loop(0, DEPTH, drain_body, 0)

    pltpu.sync_copy(data_v, out_hbm.at[wid])

  return k(flat_tensor, flat_index)


def kernel(tensor, index):
  out = _sc_gather(_to_phys(tensor), _to_phys(index))
  return _from_phys(out.reshape(TOTAL), ROWS, K)


# speedup vs baseline: 28.5416x; 1.0269x over previous
import functools

import jax
import jax.numpy as jnp
from jax import lax
from jax.experimental import pallas as pl
from jax.experimental.pallas import tpu as pltpu
from jax.experimental.pallas import tpu_sc as plsc

NUM_CORES = 2
NUM_SUBCORES = 16
NUM_WORKERS = NUM_CORES * NUM_SUBCORES
LANES = 16

ROWS = 1024
COLS = 100000
K = 200

TOTAL = ROWS * K
CHUNK = TOTAL // NUM_WORKERS
DMA_IDX = 128
N_DMA = CHUNK // DMA_IDX
VREGS_PER_DMA = DMA_IDX // LANES
DEPTH = 48


def _to_phys(x):
  r, c = x.shape
  return x.reshape(r // 128, 128, c // 8, 8).transpose(2, 0, 3, 1).reshape(-1)


def _from_phys(flat, r, c):
  return (flat.reshape(c // 8, r // 128, 8, 128)
          .transpose(1, 3, 0, 2).reshape(r, c))


def _sc_gather(flat_tensor, flat_index):
  mesh = plsc.VectorSubcoreMesh(
      core_axis_name="c", subcore_axis_name="s",
      num_cores=NUM_CORES, num_subcores=NUM_SUBCORES)

  @functools.partial(
      pl.kernel,
      out_type=jax.ShapeDtypeStruct((NUM_WORKERS, N_DMA, DMA_IDX),
                                    jnp.float32),
      mesh=mesh,
      scratch_types=[
          pltpu.VMEM((CHUNK,), jnp.int32),
          pltpu.VMEM((N_DMA, DMA_IDX), jnp.int32),
          pltpu.VMEM((N_DMA, DMA_IDX), jnp.float32),
          pltpu.SemaphoreType.DMA,
          pltpu.SemaphoreType.DMA,
      ],
  )
  def k(tensor_hbm, idx_hbm, out_hbm, idx_v, gidx_v, data_v, sem, sem2):
    wid = lax.axis_index("s") * NUM_CORES + lax.axis_index("c")
    base = wid * CHUNK
    split = DEPTH * DMA_IDX
    stage_a = pltpu.make_async_copy(
        idx_hbm.at[pl.ds(base, split)], idx_v.at[pl.ds(0, split)], sem2)
    stage_b = pltpu.make_async_copy(
        idx_hbm.at[pl.ds(base + split, CHUNK - split)],
        idx_v.at[pl.ds(split, CHUNK - split)], sem2)
    stage_a.start()
    stage_b.start()
    stage_a.wait()

    lane = lax.iota(jnp.int32, LANES)

    def compute_and_fire(c):
      s0 = c * DMA_IDX
      ixs = [idx_v[pl.ds(s0 + u * LANES, LANES)]
             for u in range(VREGS_PER_DMA)]
      ms = []
      for u in range(VREGS_PER_DMA):
        n_base = base + s0 + u * LANES
        row_part = ((lax.shift_right_logical(n_base, 10) & 7) << 10) \
            + (n_base & 127)
        ix = ixs[u]
        ms.append((lax.shift_right_logical(ix, 3) << 13)
                  + ((ix & 7) << 7) + (row_part + lane))
      for u in range(VREGS_PER_DMA):
        gidx_v[c, pl.ds(u * LANES, LANES)] = ms[u]
      pltpu.make_async_copy(
          tensor_hbm.at[gidx_v.at[c]], data_v.at[c], sem).start()

    def wait_one():
      pltpu.make_async_copy(
          tensor_hbm.at[gidx_v.at[0]], data_v.at[0], sem).wait()

    def fill_body(c, carry):
      compute_and_fire(c)
      return carry

    def steady_body(c, carry):
      wait_one()
      compute_and_fire(c)
      return carry

    def drain_body(c, carry):
      wait_one()
      return carry

    lax.fori_loop(0, DEPTH, fill_body, 0)
    stage_b.wait()
    lax.fori_loop(DEPTH, N_DMA, steady_body, 0)
    lax.fori_loop(0, DEPTH, drain_body, 0)

    pltpu.sync_copy(data_v, out_hbm.at[wid])

  return k(flat_tensor, flat_index)


def kernel(tensor, index):
  out = _sc_gather(_to_phys(tensor), _to_phys(index))
  return _from_phys(out.reshape(TOTAL), ROWS, K)


# speedup vs baseline: 28.9449x; 1.0141x over previous
import functools

import jax
import jax.numpy as jnp
from jax import lax
from jax.experimental import pallas as pl
from jax.experimental.pallas import tpu as pltpu
from jax.experimental.pallas import tpu_sc as plsc

NUM_CORES = 2
NUM_SUBCORES = 16
NUM_WORKERS = NUM_CORES * NUM_SUBCORES
LANES = 16

ROWS = 1024
COLS = 100000
K = 200

TOTAL = ROWS * K
CHUNK = TOTAL // NUM_WORKERS
DMA_IDX = 128
N_DMA = CHUNK // DMA_IDX
VREGS_PER_DMA = DMA_IDX // LANES
STAGE_SPLIT = 32


def _to_phys(x):
  r, c = x.shape
  return x.reshape(r // 128, 128, c // 8, 8).transpose(2, 0, 3, 1).reshape(-1)


def _from_phys(flat, r, c):
  return (flat.reshape(c // 8, r // 128, 8, 128)
          .transpose(1, 3, 0, 2).reshape(r, c))


def _sc_gather(flat_tensor, flat_index):
  mesh = plsc.VectorSubcoreMesh(
      core_axis_name="c", subcore_axis_name="s",
      num_cores=NUM_CORES, num_subcores=NUM_SUBCORES)

  @functools.partial(
      pl.kernel,
      out_type=jax.ShapeDtypeStruct((NUM_WORKERS, N_DMA, DMA_IDX),
                                    jnp.float32),
      mesh=mesh,
      scratch_types=[
          pltpu.VMEM((CHUNK,), jnp.int32),
          pltpu.VMEM((N_DMA, DMA_IDX), jnp.int32),
          pltpu.VMEM((N_DMA, DMA_IDX), jnp.float32),
          pltpu.SemaphoreType.DMA,
          pltpu.SemaphoreType.DMA,
      ],
  )
  def k(tensor_hbm, idx_hbm, out_hbm, idx_v, gidx_v, data_v, sem, sem2):
    wid = lax.axis_index("s") * NUM_CORES + lax.axis_index("c")
    base = wid * CHUNK
    split = STAGE_SPLIT * DMA_IDX
    stage_a = pltpu.make_async_copy(
        idx_hbm.at[pl.ds(base, split)], idx_v.at[pl.ds(0, split)], sem2)
    stage_b = pltpu.make_async_copy(
        idx_hbm.at[pl.ds(base + split, CHUNK - split)],
        idx_v.at[pl.ds(split, CHUNK - split)], sem2)
    stage_a.start()
    stage_b.start()
    stage_a.wait()

    lane = lax.iota(jnp.int32, LANES)

    def compute_and_fire(c):
      s0 = c * DMA_IDX
      ixs = [idx_v[pl.ds(s0 + u * LANES, LANES)]
             for u in range(VREGS_PER_DMA)]
      ms = []
      for u in range(VREGS_PER_DMA):
        n_base = base + s0 + u * LANES
        row_part = ((lax.shift_right_logical(n_base, 10) & 7) << 10) \
            + (n_base & 127)
        ix = ixs[u]
        ms.append((lax.shift_right_logical(ix, 3) << 13)
                  + ((ix & 7) << 7) + (row_part + lane))
      for u in range(VREGS_PER_DMA):
        gidx_v[c, pl.ds(u * LANES, LANES)] = ms[u]
      pltpu.make_async_copy(
          tensor_hbm.at[gidx_v.at[c]], data_v.at[c], sem).start()

    def fill_body(c, carry):
      compute_and_fire(c)
      return carry

    lax.fori_loop(0, STAGE_SPLIT, fill_body, 0)
    stage_b.wait()
    lax.fori_loop(STAGE_SPLIT, N_DMA, fill_body, 0)

    pltpu.make_async_copy(out_hbm.at[wid], data_v, sem).wait()

    pltpu.sync_copy(data_v, out_hbm.at[wid])

  return k(flat_tensor, flat_index)


def kernel(tensor, index):
  out = _sc_gather(_to_phys(tensor), _to_phys(index))
  return _from_phys(out.reshape(TOTAL), ROWS, K)
